# baseline (speedup 1.0000x reference)
import functools

import jax
import jax.numpy as jnp
from jax import lax
from jax.experimental import pallas as pl
from jax.experimental.pallas import tpu as pltpu
from jax.experimental.pallas import tpu_sc as plsc

_NUM_CORES = 2
_NUM_SUBCORES = 16
_NUM_WORKERS = _NUM_CORES * _NUM_SUBCORES

_BLOCK = 512
_CPB = 128


_SKEW = 137


def _transpose_block(tin, tmid, tout, col_consts):
    @plsc.parallel_loop(0, 64, unroll=16)
    def scatter(j):
        for l0 in range(8):
            v = tin[j, pl.ds(l0 * 16, 16)]
            plsc.store_scatter(tmid, [col_consts[l0] + j], v)

    @plsc.parallel_loop(0, 128, unroll=16)
    def compact(l):
        for j0 in range(4):
            tout[pl.ds(l * 64 + j0 * 16, 16)] = tmid[pl.ds(l * _SKEW + j0 * 16, 16)]


@jax.jit
def _table_linearize(wt, tail_flat):
    d, n_rows = wt.shape
    n_main = (n_rows // _CPB) * _CPB
    nblk = n_main // _CPB
    per_w = nblk // _NUM_WORKERS
    extra = nblk - per_w * _NUM_WORKERS

    @functools.partial(
        pl.kernel,
        out_type=jax.ShapeDtypeStruct((n_rows * d,), jnp.float32),
        mesh=plsc.VectorSubcoreMesh(core_axis_name="c", subcore_axis_name="s"),
        scratch_types=[
            pltpu.VMEM((d, _CPB), jnp.float32),
            pltpu.VMEM((d, _CPB), jnp.float32),
            pltpu.VMEM((_CPB * _SKEW,), jnp.float32),
            pltpu.VMEM((_CPB * _SKEW,), jnp.float32),
            pltpu.VMEM((_CPB * d,), jnp.float32),
            pltpu.VMEM((_CPB * d,), jnp.float32),
            pltpu.SemaphoreType.DMA,
            pltpu.SemaphoreType.DMA,
            pltpu.SemaphoreType.DMA,
            pltpu.SemaphoreType.DMA,
        ],
        compiler_params=pltpu.CompilerParams(needs_layout_passes=False),
    )
    def k(wt_hbm, tail_hbm, out_hbm, tin0, tin1, tmid0, tmid1, tout0, tout1,
          si0, si1, so0, so1):
        wid = lax.axis_index("s") * _NUM_CORES + lax.axis_index("c")
        lanes = lax.iota(jnp.int32, 16)
        col_consts = [(lanes + l0 * 16) * _SKEW for l0 in range(8)]

        n_tail = (n_rows - n_main) * d

        @pl.when(wid == _NUM_WORKERS - 1)
        def _():
            pltpu.sync_copy(tail_hbm, tout0.at[pl.ds(0, n_tail)])
            pltpu.sync_copy(
                tout0.at[pl.ds(0, n_tail)],
                out_hbm.at[pl.ds(n_main * d, n_tail)],
            )

        def stage_in(c, tin, sem):
            pltpu.make_async_copy(
                wt_hbm.at[:, pl.ds(c * _CPB, _CPB)], tin, sem
            ).start()

        def wait_in(tin, sem):
            pltpu.make_async_copy(
                wt_hbm.at[:, pl.ds(0, _CPB)], tin, sem
            ).wait()

        def stage_out(c, tout, sem):
            pltpu.make_async_copy(
                tout, out_hbm.at[pl.ds(c * _CPB * d, _CPB * d)], sem
            ).start()

        def wait_out(tout, sem):
            pltpu.make_async_copy(
                tout, out_hbm.at[pl.ds(0, _CPB * d)], sem
            ).wait()

        def block_c(g):
            return g * _NUM_WORKERS + wid

        stage_in(block_c(0), tin0, si0)
        stage_in(block_c(1), tin1, si1)

        def body(it, carry):
            g0 = 2 * it
            g1 = g0 + 1
            wait_in(tin0, si0)
            _transpose_block(tin0, tmid0, tout0, col_consts)

            @pl.when(g0 + 2 < per_w)
            def _():
                stage_in(block_c(g0 + 2), tin0, si0)

            @pl.when(g0 > 0)
            def _():
                wait_out(tout0, so0)

            stage_out(block_c(g0), tout0, so0)

            wait_in(tin1, si1)
            _transpose_block(tin1, tmid1, tout1, col_consts)

            @pl.when(g1 + 2 < per_w)
            def _():
                stage_in(block_c(g1 + 2), tin1, si1)

            @pl.when(g1 > 1)
            def _():
                wait_out(tout1, so1)

            stage_out(block_c(g1), tout1, so1)
            return carry

        lax.fori_loop(0, per_w // 2, body, 0)

        @pl.when(wid < extra)
        def _():
            c = per_w * _NUM_WORKERS + wid
            pltpu.sync_copy(wt_hbm.at[:, pl.ds(c * _CPB, _CPB)], tin0)
            wait_out(tout0, so0)
            _transpose_block(tin0, tmid0, tout0, col_consts)
            stage_out(c, tout0, so0)

        wait_out(tout0, so0)
        wait_out(tout1, so1)

    return k(wt, tail_flat)


@functools.partial(jax.jit, static_argnames=("rows_per_worker",))
def _sc_gather(flat_ids, table, rows_per_worker):
    n, d = flat_ids.shape[0], table.shape[1]
    num_blocks = rows_per_worker // _BLOCK

    @functools.partial(
        pl.kernel,
        out_type=jax.ShapeDtypeStruct((n, 2 * d), jnp.float32),
        mesh=plsc.VectorSubcoreMesh(core_axis_name="c", subcore_axis_name="s"),
        scratch_types=[
            pltpu.VMEM((rows_per_worker,), jnp.int32),
            pltpu.VMEM((_BLOCK, d), jnp.float32),
            pltpu.VMEM((_BLOCK, d), jnp.float32),
            pltpu.SemaphoreType.DMA,
            pltpu.SemaphoreType.DMA,
            pltpu.SemaphoreType.DMA,
            pltpu.SemaphoreType.DMA,
        ],
        compiler_params=pltpu.CompilerParams(use_tc_tiling_on_sc=False),
    )
    def k(ids_hbm, table_hbm, out_hbm, idx_v, rows0, rows1, sg0, sg1, so0, so1):
        wid = lax.axis_index("s") * _NUM_CORES + lax.axis_index("c")
        base = wid * rows_per_worker
        pltpu.sync_copy(ids_hbm.at[pl.ds(base, rows_per_worker)], idx_v)

        def gather(g, rows, sem):
            pltpu.make_async_copy(
                table_hbm.at[idx_v.at[pl.ds(g * _BLOCK, _BLOCK)]], rows, sem
            ).start()

        def writeback(g, rows, sem):
            pltpu.make_async_copy(
                rows,
                out_hbm.at[pl.ds(base + g * _BLOCK, _BLOCK), pl.ds(0, d)],
                sem,
            ).start()

        def wait_gather(rows, sem):
            pltpu.make_async_copy(
                table_hbm.at[idx_v.at[pl.ds(0, _BLOCK)]], rows, sem
            ).wait()

        def wait_writeback(rows, sem):
            pltpu.make_async_copy(
                rows, out_hbm.at[pl.ds(base, _BLOCK), pl.ds(0, d)], sem
            ).wait()

        gather(0, rows0, sg0)
        gather(1, rows1, sg1)

        def body(it, carry):
            g0 = 2 * it
            g1 = g0 + 1
            wait_gather(rows0, sg0)
            writeback(g0, rows0, so0)
            wait_gather(rows1, sg1)
            writeback(g1, rows1, so1)

            @pl.when(g0 + 2 < num_blocks)
            def _():
                wait_writeback(rows0, so0)
                gather(g0 + 2, rows0, sg0)
                wait_writeback(rows1, so1)
                gather(g1 + 2, rows1, sg1)

            return carry

        lax.fori_loop(0, num_blocks // 2, body, 0)

        wait_writeback(rows0, so0)
        wait_writeback(rows1, so1)

    return k(flat_ids, table)


def kernel(token_ids, weight):
    b, s = token_ids.shape
    v, d = weight.shape
    flat = token_ids.reshape(-1).astype(jnp.int32)
    n = flat.shape[0]
    n_main = (v // _CPB) * _CPB
    table1d = _table_linearize(weight.T, weight[n_main:].reshape(-1))
    table = table1d.reshape(v, d)
    rows_per_worker = n // _NUM_WORKERS
    out128 = _sc_gather(flat, table, rows_per_worker)
    return out128[:, :d].reshape(b, s, d)


# speedup vs baseline: 1.0042x; 1.0042x over previous
import functools

import jax
import jax.numpy as jnp
from jax import lax
from jax.experimental import pallas as pl
from jax.experimental.pallas import tpu as pltpu
from jax.experimental.pallas import tpu_sc as plsc

_NUM_CORES = 2
_NUM_SUBCORES = 16
_NUM_WORKERS = _NUM_CORES * _NUM_SUBCORES

_BLOCK = 512
_CPB = 128


_SKEW = 137


def _transpose_block(tin, tmid, tout, col_consts):
    @plsc.parallel_loop(0, 64, unroll=8)
    def scatter(j):
        for l0 in range(8):
            v = tin[j, pl.ds(l0 * 16, 16)]
            plsc.store_scatter(tmid, [col_consts[l0] + j], v)

    @plsc.parallel_loop(0, 128, unroll=8)
    def compact(l):
        for j0 in range(4):
            tout[pl.ds(l * 64 + j0 * 16, 16)] = tmid[pl.ds(l * _SKEW + j0 * 16, 16)]


@jax.jit
def _table_linearize(wt, tail_flat):
    d, n_rows = wt.shape
    n_main = (n_rows // _CPB) * _CPB
    nblk = n_main // _CPB
    per_w = nblk // _NUM_WORKERS
    extra = nblk - per_w * _NUM_WORKERS

    @functools.partial(
        pl.kernel,
        out_type=jax.ShapeDtypeStruct((n_rows * d,), jnp.float32),
        mesh=plsc.VectorSubcoreMesh(core_axis_name="c", subcore_axis_name="s"),
        scratch_types=[
            pltpu.VMEM((d, _CPB), jnp.float32),
            pltpu.VMEM((d, _CPB), jnp.float32),
            pltpu.VMEM((_CPB * _SKEW,), jnp.float32),
            pltpu.VMEM((_CPB * _SKEW,), jnp.float32),
            pltpu.VMEM((_CPB * d,), jnp.float32),
            pltpu.VMEM((_CPB * d,), jnp.float32),
            pltpu.SemaphoreType.DMA,
            pltpu.SemaphoreType.DMA,
            pltpu.SemaphoreType.DMA,
            pltpu.SemaphoreType.DMA,
        ],
        compiler_params=pltpu.CompilerParams(needs_layout_passes=False),
    )
    def k(wt_hbm, tail_hbm, out_hbm, tin0, tin1, tmid0, tmid1, tout0, tout1,
          si0, si1, so0, so1):
        wid = lax.axis_index("s") * _NUM_CORES + lax.axis_index("c")
        lanes = lax.iota(jnp.int32, 16)
        col_consts = [(lanes + l0 * 16) * _SKEW for l0 in range(8)]

        n_tail = (n_rows - n_main) * d

        @pl.when(wid == _NUM_WORKERS - 1)
        def _():
            pltpu.sync_copy(tail_hbm, tout0.at[pl.ds(0, n_tail)])
            pltpu.sync_copy(
                tout0.at[pl.ds(0, n_tail)],
                out_hbm.at[pl.ds(n_main * d, n_tail)],
            )

        def stage_in(c, tin, sem):
            pltpu.make_async_copy(
                wt_hbm.at[:, pl.ds(c * _CPB, _CPB)], tin, sem
            ).start()

        def wait_in(tin, sem):
            pltpu.make_async_copy(
                wt_hbm.at[:, pl.ds(0, _CPB)], tin, sem
            ).wait()

        def stage_out(c, tout, sem):
            pltpu.make_async_copy(
                tout, out_hbm.at[pl.ds(c * _CPB * d, _CPB * d)], sem
            ).start()

        def wait_out(tout, sem):
            pltpu.make_async_copy(
                tout, out_hbm.at[pl.ds(0, _CPB * d)], sem
            ).wait()

        def block_c(g):
            return g * _NUM_WORKERS + wid

        stage_in(block_c(0), tin0, si0)
        stage_in(block_c(1), tin1, si1)

        def body(it, carry):
            g0 = 2 * it
            g1 = g0 + 1
            wait_in(tin0, si0)
            _transpose_block(tin0, tmid0, tout0, col_consts)

            @pl.when(g0 + 2 < per_w)
            def _():
                stage_in(block_c(g0 + 2), tin0, si0)

            @pl.when(g0 > 0)
            def _():
                wait_out(tout0, so0)

            stage_out(block_c(g0), tout0, so0)

            wait_in(tin1, si1)
            _transpose_block(tin1, tmid1, tout1, col_consts)

            @pl.when(g1 + 2 < per_w)
            def _():
                stage_in(block_c(g1 + 2), tin1, si1)

            @pl.when(g1 > 1)
            def _():
                wait_out(tout1, so1)

            stage_out(block_c(g1), tout1, so1)
            return carry

        lax.fori_loop(0, per_w // 2, body, 0)

        @pl.when(wid < extra)
        def _():
            c = per_w * _NUM_WORKERS + wid
            pltpu.sync_copy(wt_hbm.at[:, pl.ds(c * _CPB, _CPB)], tin0)
            wait_out(tout0, so0)
            _transpose_block(tin0, tmid0, tout0, col_consts)
            stage_out(c, tout0, so0)

        wait_out(tout0, so0)
        wait_out(tout1, so1)

    return k(wt, tail_flat)


@functools.partial(jax.jit, static_argnames=("rows_per_worker",))
def _sc_gather(flat_ids, table, rows_per_worker):
    n, d = flat_ids.shape[0], table.shape[1]
    num_blocks = rows_per_worker // _BLOCK

    @functools.partial(
        pl.kernel,
        out_type=jax.ShapeDtypeStruct((n, 2 * d), jnp.float32),
        mesh=plsc.VectorSubcoreMesh(core_axis_name="c", subcore_axis_name="s"),
        scratch_types=[
            pltpu.VMEM((rows_per_worker,), jnp.int32),
            pltpu.VMEM((_BLOCK, d), jnp.float32),
            pltpu.VMEM((_BLOCK, d), jnp.float32),
            pltpu.SemaphoreType.DMA,
            pltpu.SemaphoreType.DMA,
            pltpu.SemaphoreType.DMA,
            pltpu.SemaphoreType.DMA,
        ],
        compiler_params=pltpu.CompilerParams(use_tc_tiling_on_sc=False),
    )
    def k(ids_hbm, table_hbm, out_hbm, idx_v, rows0, rows1, sg0, sg1, so0, so1):
        wid = lax.axis_index("s") * _NUM_CORES + lax.axis_index("c")
        base = wid * rows_per_worker
        pltpu.sync_copy(ids_hbm.at[pl.ds(base, rows_per_worker)], idx_v)

        def gather(g, rows, sem):
            pltpu.make_async_copy(
                table_hbm.at[idx_v.at[pl.ds(g * _BLOCK, _BLOCK)]], rows, sem
            ).start()

        def writeback(g, rows, sem):
            pltpu.make_async_copy(
                rows,
                out_hbm.at[pl.ds(base + g * _BLOCK, _BLOCK), pl.ds(0, d)],
                sem,
            ).start()

        def wait_gather(rows, sem):
            pltpu.make_async_copy(
                table_hbm.at[idx_v.at[pl.ds(0, _BLOCK)]], rows, sem
            ).wait()

        def wait_writeback(rows, sem):
            pltpu.make_async_copy(
                rows, out_hbm.at[pl.ds(base, _BLOCK), pl.ds(0, d)], sem
            ).wait()

        gather(0, rows0, sg0)
        gather(1, rows1, sg1)

        def body(it, carry):
            g0 = 2 * it
            g1 = g0 + 1
            wait_gather(rows0, sg0)
            writeback(g0, rows0, so0)
            wait_gather(rows1, sg1)
            writeback(g1, rows1, so1)

            @pl.when(g0 + 2 < num_blocks)
            def _():
                wait_writeback(rows0, so0)
                gather(g0 + 2, rows0, sg0)
                wait_writeback(rows1, so1)
                gather(g1 + 2, rows1, sg1)

            return carry

        lax.fori_loop(0, num_blocks // 2, body, 0)

        wait_writeback(rows0, so0)
        wait_writeback(rows1, so1)

    return k(flat_ids, table)


def kernel(token_ids, weight):
    b, s = token_ids.shape
    v, d = weight.shape
    flat = token_ids.reshape(-1).astype(jnp.int32)
    n = flat.shape[0]
    n_main = (v // _CPB) * _CPB
    table1d = _table_linearize(weight.T, weight[n_main:].reshape(-1))
    table = table1d.reshape(v, d)
    rows_per_worker = n // _NUM_WORKERS
    out128 = _sc_gather(flat, table, rows_per_worker)
    return out128[:, :d].reshape(b, s, d)
